# baseline (speedup 1.0000x reference)
import functools

import jax
import jax.numpy as jnp
from jax.experimental import pallas as pl
from jax.experimental.pallas import tpu as pltpu


def _fused_body(w1_ref, w2_ref, x0_ref, x1_ref, o_ref, *, inv_hw, C):
    x0 = x0_ref[...]
    x1 = x1_ref[...]

    pooled = jnp.sum(x0 + x1, axis=1) * inv_hw

    hid = jnp.maximum(
        jnp.dot(pooled, w1_ref[...], preferred_element_type=jnp.float32), 0.0)
    logits = jnp.dot(hid, w2_ref[...], preferred_element_type=jnp.float32)

    a0 = jax.nn.sigmoid(logits[:, :C] - logits[:, C:])

    o_ref[...] = x1 + a0[:, None, :] * (x0 - x1)


def kernel(feat0, feat1, w1, w2):
    B, C, H, W = feat0.shape
    HW = H * W
    d = w1.shape[1]

    x0 = feat0.transpose(0, 2, 3, 1).reshape(B, HW, C)
    x1 = feat1.transpose(0, 2, 3, 1).reshape(B, HW, C)

    bB = 2
    n_chunks = B // bB
    n_split = 2 if n_chunks % 2 == 0 else 1
    per_core = n_chunks // n_split

    feat_spec = pl.BlockSpec((bB, HW, C), lambda c, i: (c * per_core + i, 0, 0))
    cost = pl.CostEstimate(
        flops=int(5 * B * C * HW + 2 * B * d * C * 3),
        transcendentals=int(B * C),
        bytes_accessed=int(3 * B * C * HW * 4 + (C * d + d * 2 * C) * 4))

    out = pl.pallas_call(
        functools.partial(_fused_body, inv_hw=1.0 / HW, C=C),
        out_shape=jax.ShapeDtypeStruct((B, HW, C), feat0.dtype),
        grid=(n_split, per_core),
        in_specs=[
            pl.BlockSpec((C, d), lambda c, i: (0, 0)),
            pl.BlockSpec((d, 2 * C), lambda c, i: (0, 0)),
            feat_spec,
            feat_spec,
        ],
        out_specs=feat_spec,
        compiler_params=pltpu.CompilerParams(
            dimension_semantics=("parallel", "arbitrary"),
            vmem_limit_bytes=int(60 * 1024 * 1024)),
        cost_estimate=cost,
    )(w1, w2, x0, x1)

    return out.reshape(B, H, W, C).transpose(0, 3, 1, 2)


# speedup vs baseline: 1.0441x; 1.0441x over previous
import functools

import jax
import jax.numpy as jnp
from jax.experimental import pallas as pl
from jax.experimental.pallas import tpu as pltpu


def _fused_body(w1_ref, w2_ref, x0_ref, x1_ref, o_ref, *, inv_hw, C):
    x0 = x0_ref[...]
    x1 = x1_ref[...]

    pooled = (jnp.sum(x0, axis=1) + jnp.sum(x1, axis=1)) * inv_hw

    hid = jnp.maximum(
        jnp.dot(pooled, w1_ref[...], preferred_element_type=jnp.float32), 0.0)
    logits = jnp.dot(hid, w2_ref[...], preferred_element_type=jnp.float32)

    a0 = jax.nn.sigmoid(logits[:, :C] - logits[:, C:])

    o_ref[...] = x1 + a0[:, None, :] * (x0 - x1)


def kernel(feat0, feat1, w1, w2):
    B, C, H, W = feat0.shape
    HW = H * W
    d = w1.shape[1]

    x0 = feat0.transpose(0, 2, 3, 1).reshape(B, HW, C)
    x1 = feat1.transpose(0, 2, 3, 1).reshape(B, HW, C)

    bB = 4
    n_chunks = B // bB
    n_split = 2 if n_chunks % 2 == 0 else 1
    per_core = n_chunks // n_split

    feat_spec = pl.BlockSpec((bB, HW, C), lambda c, i: (c * per_core + i, 0, 0))
    cost = pl.CostEstimate(
        flops=int(5 * B * C * HW + 2 * B * d * C * 3),
        transcendentals=int(B * C),
        bytes_accessed=int(3 * B * C * HW * 4 + (C * d + d * 2 * C) * 4))

    out = pl.pallas_call(
        functools.partial(_fused_body, inv_hw=1.0 / HW, C=C),
        out_shape=jax.ShapeDtypeStruct((B, HW, C), feat0.dtype),
        grid=(n_split, per_core),
        in_specs=[
            pl.BlockSpec((C, d), lambda c, i: (0, 0)),
            pl.BlockSpec((d, 2 * C), lambda c, i: (0, 0)),
            feat_spec,
            feat_spec,
        ],
        out_specs=feat_spec,
        compiler_params=pltpu.CompilerParams(
            dimension_semantics=("parallel", "arbitrary"),
            vmem_limit_bytes=int(60 * 1024 * 1024)),
        cost_estimate=cost,
    )(w1, w2, x0, x1)

    return out.reshape(B, H, W, C).transpose(0, 3, 1, 2)


# speedup vs baseline: 1.0470x; 1.0027x over previous
import functools

import jax
import jax.numpy as jnp
from jax.experimental import pallas as pl
from jax.experimental.pallas import tpu as pltpu


def _fused_body(w1_ref, w2_ref, x0_ref, x1_ref, o_ref, *, inv_hw, C):
    x0 = x0_ref[...]
    x1 = x1_ref[...]

    pooled = (jnp.sum(x0, axis=1) + jnp.sum(x1, axis=1)) * inv_hw

    hid = jnp.maximum(
        jnp.dot(pooled, w1_ref[...], preferred_element_type=jnp.float32), 0.0)
    logits = jnp.dot(hid, w2_ref[...], preferred_element_type=jnp.float32)

    a0 = jax.nn.sigmoid(logits[:, :C] - logits[:, C:])

    o_ref[...] = x1 + a0[:, None, :] * (x0 - x1)


def kernel(feat0, feat1, w1, w2):
    B, C, H, W = feat0.shape
    HW = H * W
    d = w1.shape[1]

    x0 = feat0.transpose(0, 2, 3, 1).reshape(B, HW, C)
    x1 = feat1.transpose(0, 2, 3, 1).reshape(B, HW, C)

    bB = 4
    n_chunks = B // bB
    n_split = 2 if n_chunks % 2 == 0 else 1
    per_core = n_chunks // n_split

    feat_spec = pl.BlockSpec((bB, HW, C), lambda c, i: (i * n_split + c, 0, 0))
    cost = pl.CostEstimate(
        flops=int(5 * B * C * HW + 2 * B * d * C * 3),
        transcendentals=int(B * C),
        bytes_accessed=int(3 * B * C * HW * 4 + (C * d + d * 2 * C) * 4))

    out = pl.pallas_call(
        functools.partial(_fused_body, inv_hw=1.0 / HW, C=C),
        out_shape=jax.ShapeDtypeStruct((B, HW, C), feat0.dtype),
        grid=(n_split, per_core),
        in_specs=[
            pl.BlockSpec((C, d), lambda c, i: (0, 0)),
            pl.BlockSpec((d, 2 * C), lambda c, i: (0, 0)),
            feat_spec,
            feat_spec,
        ],
        out_specs=feat_spec,
        compiler_params=pltpu.CompilerParams(
            dimension_semantics=("parallel", "arbitrary"),
            vmem_limit_bytes=int(60 * 1024 * 1024)),
        cost_estimate=cost,
    )(w1, w2, x0, x1)

    return out.reshape(B, H, W, C).transpose(0, 3, 1, 2)
